# baseline (speedup 1.0000x reference)
import jax
import jax.numpy as jnp
from jax.experimental import pallas as pl


def _copy_body(x_ref, o_ref):
    o_ref[...] = x_ref[...]


def _pick_chunk(n_cols):
    for chunk in (128000, 64000, 32000, 12800, 6400, 1280, 128):
        if n_cols % chunk == 0:
            return chunk
    return None


def kernel(edge_index):
    two, n_cols = edge_index.shape
    chunk = _pick_chunk(n_cols)
    if chunk is None:
        chunk = n_cols
    grid = n_cols // chunk
    return pl.pallas_call(
        _copy_body,
        grid=(grid,),
        in_specs=[pl.BlockSpec((two, chunk), lambda i: (0, i))],
        out_specs=pl.BlockSpec((two, chunk), lambda i: (0, i)),
        out_shape=jax.ShapeDtypeStruct(edge_index.shape, edge_index.dtype),
    )(edge_index)


# speedup vs baseline: 1.5941x; 1.5941x over previous
import jax
import jax.numpy as jnp
from jax.experimental import pallas as pl


def _copy_body(x_ref, o_ref):
    o_ref[...] = x_ref[...]


def _pick_chunk(n_cols):
    for chunk in (1280000, 640000, 128000, 64000, 32000, 12800, 6400, 1280, 128):
        if n_cols % chunk == 0:
            return chunk
    return None


def kernel(edge_index):
    two, n_cols = edge_index.shape
    chunk = _pick_chunk(n_cols)
    if chunk is None:
        chunk = n_cols
    grid = n_cols // chunk
    return pl.pallas_call(
        _copy_body,
        grid=(grid,),
        in_specs=[pl.BlockSpec((two, chunk), lambda i: (0, i))],
        out_specs=pl.BlockSpec((two, chunk), lambda i: (0, i)),
        out_shape=jax.ShapeDtypeStruct(edge_index.shape, edge_index.dtype),
    )(edge_index)
